# Initial kernel scaffold; baseline (speedup 1.0000x reference)
import jax
import jax.numpy as jnp
from jax.experimental import pallas as pl


def kernel(atom_features, bond_features, bond_index, molecule_features, atom_to_molecule, W_i, W_h, W_o, b_o, W1, b1, W2, b2):
    raise NotImplementedError("write your pallas kernel here")


# speedup vs baseline: 1.0212x; 1.0212x over previous
import functools

import jax
import jax.numpy as jnp
from jax import lax
from jax.experimental import pallas as pl
from jax.experimental.pallas import tpu as pltpu
from jax.experimental.pallas import tpu_sc as plsc

N_ATOMS = 10000
N_BONDS = 320000
HIDDEN = 128
DEPTH = 3

NC = 2
NS = 16
NW = NC * NS
EPW = N_BONDS // NW
CHUNK = 80
NCHUNK = EPW // CHUNK
A_PAD = 10240

ATOM_BLK = 1024
EDGE_BLK = 512
MOL_PAD = 256


def _sc_mesh():
    return plsc.VectorSubcoreMesh(
        core_axis_name="c", subcore_axis_name="s",
        num_cores=NC, num_subcores=NS)


def _gather_body(table_hbm, idx_hbm, out_hbm, idx_v, rows_v, sem):
    c = lax.axis_index("c")
    s = lax.axis_index("s")
    w = c * NS + s
    pltpu.sync_copy(idx_hbm.at[w], idx_v)

    def body(j, carry):
        pltpu.async_copy(table_hbm.at[idx_v.at[j]], rows_v, sem).wait()
        pltpu.sync_copy(rows_v, out_hbm.at[pl.ds(w * EPW + j * CHUNK, CHUNK)])
        return carry

    lax.fori_loop(0, NCHUNK, body, 0)


@functools.cache
def _build_gather():
    return pl.kernel(
        _gather_body,
        out_type=jax.ShapeDtypeStruct((N_BONDS, HIDDEN), jnp.float32),
        mesh=_sc_mesh(),
        scratch_types=[
            pltpu.VMEM((NCHUNK, CHUNK), jnp.int32),
            pltpu.VMEM((CHUNK, HIDDEN), jnp.float32),
            pltpu.SemaphoreType.DMA,
        ],
    )


def _gather_rows(table, idx3):
    return _build_gather()(table, idx3)


def _segsum_body(h_hbm, idx_hbm, zeros_hbm, out_hbm, idx_v, rows_v, acc_sh):
    c = lax.axis_index("c")
    s = lax.axis_index("s")
    w = c * NS + s
    rows_per_tile = A_PAD // NS
    pltpu.sync_copy(zeros_hbm.at[pl.ds(s * rows_per_tile, rows_per_tile)],
                    acc_sh.at[pl.ds(s * rows_per_tile, rows_per_tile)])
    pltpu.sync_copy(idx_hbm.at[w], idx_v)
    plsc.subcore_barrier()

    def body(j, carry):
        pltpu.sync_copy(h_hbm.at[pl.ds(w * EPW + j * CHUNK, CHUNK)], rows_v)
        pltpu.sync_copy(rows_v, acc_sh.at[idx_v.at[j]], add=True)
        return carry

    lax.fori_loop(0, NCHUNK, body, 0)
    plsc.subcore_barrier()
    pltpu.sync_copy(acc_sh.at[pl.ds(s * rows_per_tile, rows_per_tile)],
                    out_hbm.at[c, pl.ds(s * rows_per_tile, rows_per_tile)])


@functools.cache
def _build_segsum():
    return pl.kernel(
        _segsum_body,
        out_type=jax.ShapeDtypeStruct((NC, A_PAD, HIDDEN), jnp.float32),
        mesh=_sc_mesh(),
        scratch_types=[
            pltpu.VMEM((NCHUNK, CHUNK), jnp.int32),
            pltpu.VMEM((CHUNK, HIDDEN), jnp.float32),
            pltpu.VMEM_SHARED((A_PAD, HIDDEN), jnp.float32),
        ],
    )


def _segment_sum(h, idx3, zeros):
    return _build_segsum()(h, idx3, zeros)


def _mm_small_body(x_ref, w_ref, o_ref):
    o_ref[...] = jnp.dot(x_ref[...], w_ref[...],
                         preferred_element_type=jnp.float32)


def _mm_small(x, w):
    grid = A_PAD // ATOM_BLK
    return pl.pallas_call(
        _mm_small_body,
        grid=(grid,),
        in_specs=[pl.BlockSpec((ATOM_BLK, HIDDEN), lambda i: (i, 0)),
                  pl.BlockSpec((HIDDEN, HIDDEN), lambda i: (0, 0))],
        out_specs=pl.BlockSpec((ATOM_BLK, HIDDEN), lambda i: (i, 0)),
        out_shape=jax.ShapeDtypeStruct((A_PAD, HIDDEN), jnp.float32),
    )(x, w)


def _mm_partial_body(p_ref, w_ref, o_ref):
    x = p_ref[0] + p_ref[1]
    o_ref[...] = jnp.dot(x, w_ref[...], preferred_element_type=jnp.float32)


def _mm_partials(p, w):
    grid = A_PAD // ATOM_BLK
    return pl.pallas_call(
        _mm_partial_body,
        grid=(grid,),
        in_specs=[pl.BlockSpec((NC, ATOM_BLK, HIDDEN), lambda i: (0, i, 0)),
                  pl.BlockSpec((HIDDEN, HIDDEN), lambda i: (0, 0))],
        out_specs=pl.BlockSpec((ATOM_BLK, HIDDEN), lambda i: (i, 0)),
        out_shape=jax.ShapeDtypeStruct((A_PAD, HIDDEN), jnp.float32),
    )(p, w)


def _h0_body(g_ref, bf_ref, w_ref, o_ref):
    acc = g_ref[...] + jnp.dot(bf_ref[...], w_ref[...],
                               preferred_element_type=jnp.float32)
    o_ref[...] = jnp.maximum(acc, 0.0)


def _h0_kernel(g0, bfr, w_ib):
    grid = N_BONDS // EDGE_BLK
    bd = bfr.shape[1]
    return pl.pallas_call(
        _h0_body,
        grid=(grid,),
        in_specs=[pl.BlockSpec((EDGE_BLK, HIDDEN), lambda i: (i, 0)),
                  pl.BlockSpec((EDGE_BLK, bd), lambda i: (i, 0)),
                  pl.BlockSpec((bd, HIDDEN), lambda i: (0, 0))],
        out_specs=pl.BlockSpec((EDGE_BLK, HIDDEN), lambda i: (i, 0)),
        out_shape=jax.ShapeDtypeStruct((N_BONDS, HIDDEN), jnp.float32),
    )(g0, bfr, w_ib)


def _update_body(h0_ref, g_ref, h_ref, w_ref, o_ref):
    hw = jnp.dot(h_ref[...], w_ref[...], preferred_element_type=jnp.float32)
    o_ref[...] = jnp.maximum(h0_ref[...] + g_ref[...] - hw, 0.0)


def _update_kernel(h0r, g, hr, w_h):
    grid = N_BONDS // EDGE_BLK
    spec = pl.BlockSpec((EDGE_BLK, HIDDEN), lambda i: (i, 0))
    return pl.pallas_call(
        _update_body,
        grid=(grid,),
        in_specs=[spec, spec, spec,
                  pl.BlockSpec((HIDDEN, HIDDEN), lambda i: (0, 0))],
        out_specs=pl.BlockSpec((EDGE_BLK, HIDDEN), lambda i: (i, 0)),
        out_shape=jax.ShapeDtypeStruct((N_BONDS, HIDDEN), jnp.float32),
    )(h0r, g, hr, w_h)


def _final_body(af_ref, p_ref, a2m_ref, woa_ref, wob_ref, bo_ref,
                mf_ref, w1a_ref, w1b_ref, b1_ref, w2_ref, b2_ref,
                o_ref, msum_ref, cnt_ref):
    i = pl.program_id(0)

    @pl.when(i == 0)
    def _():
        msum_ref[...] = jnp.zeros_like(msum_ref)
        cnt_ref[...] = jnp.zeros_like(cnt_ref)

    m = p_ref[0] + p_ref[1]
    ah = jnp.dot(af_ref[...], woa_ref[...], preferred_element_type=jnp.float32)
    ah = ah + jnp.dot(m, wob_ref[...], preferred_element_type=jnp.float32)
    ah = jnp.maximum(ah + bo_ref[...], 0.0)
    ids = a2m_ref[0]
    onehot = (lax.broadcasted_iota(jnp.int32, (MOL_PAD, ATOM_BLK), 0)
              == ids).astype(jnp.float32)
    msum_ref[...] += jnp.dot(onehot, ah, preferred_element_type=jnp.float32)
    cnt_ref[...] += jnp.dot(onehot,
                            jnp.ones((ATOM_BLK, HIDDEN), jnp.float32),
                            preferred_element_type=jnp.float32)

    @pl.when(i == pl.num_programs(0) - 1)
    def _():
        mol = msum_ref[...] / jnp.maximum(cnt_ref[...], 1.0)
        x = jnp.dot(mol, w1a_ref[...], preferred_element_type=jnp.float32)
        x = x + jnp.dot(mf_ref[...], w1b_ref[...],
                        preferred_element_type=jnp.float32)
        x = jax.nn.sigmoid(x + b1_ref[...])
        o = jnp.dot(x, w2_ref[...], preferred_element_type=jnp.float32)
        o_ref[...] = jax.nn.sigmoid(o + b2_ref[...])


def _final_kernel(af, partials, a2m3, w_oa, w_ob, b_o2, mfp, w1a, w1bp,
                  b12, w2p, b22, ffn_hidden):
    grid = A_PAD // ATOM_BLK
    const = lambda shape: pl.BlockSpec(shape, lambda i: tuple(0 for _ in shape))
    return pl.pallas_call(
        _final_body,
        grid=(grid,),
        in_specs=[
            pl.BlockSpec((ATOM_BLK, HIDDEN), lambda i: (i, 0)),
            pl.BlockSpec((NC, ATOM_BLK, HIDDEN), lambda i: (0, i, 0)),
            pl.BlockSpec((1, 1, ATOM_BLK), lambda i: (i, 0, 0)),
            const((HIDDEN, HIDDEN)),
            const((HIDDEN, HIDDEN)),
            const((1, HIDDEN)),
            const((MOL_PAD, MOL_PAD)),
            const((HIDDEN, ffn_hidden)),
            const((MOL_PAD, ffn_hidden)),
            const((1, ffn_hidden)),
            const((ffn_hidden, HIDDEN)),
            const((1, HIDDEN)),
        ],
        out_specs=pl.BlockSpec((MOL_PAD, HIDDEN), lambda i: (0, 0)),
        out_shape=jax.ShapeDtypeStruct((MOL_PAD, HIDDEN), jnp.float32),
        scratch_shapes=[pltpu.VMEM((MOL_PAD, HIDDEN), jnp.float32),
                        pltpu.VMEM((MOL_PAD, HIDDEN), jnp.float32)],
    )(af, partials, a2m3, w_oa, w_ob, b_o2, mfp, w1a, w1bp, b12, w2p, b22)


def _swap_pairs(x):
    shp = x.shape
    return x.reshape((shp[0] // 2, 2) + shp[1:])[:, ::-1].reshape(shp)


def kernel(atom_features, bond_features, bond_index, molecule_features,
           atom_to_molecule, W_i, W_h, W_o, b_o, W1, b1, W2, b2):
    f32 = jnp.float32
    n_mol, feat_dim = molecule_features.shape
    ffn_hidden = W1.shape[1]

    src = bond_index[0].astype(jnp.int32)
    dst = bond_index[1].astype(jnp.int32)
    srcr = _swap_pairs(src).reshape(NW, NCHUNK, CHUNK)
    dstr = _swap_pairs(dst).reshape(NW, NCHUNK, CHUNK)
    bfr = _swap_pairs(bond_features)
    a2m_pad = jnp.full((A_PAD,), MOL_PAD - 1, jnp.int32).at[:N_ATOMS].set(
        atom_to_molecule.astype(jnp.int32))
    a2m3 = a2m_pad.reshape(A_PAD // ATOM_BLK, 1, ATOM_BLK)
    zeros_atoms = jnp.zeros((A_PAD, HIDDEN), f32)
    af_pad = jnp.zeros((A_PAD, atom_features.shape[1]), f32).at[:N_ATOMS].set(
        atom_features)

    atom_dim = atom_features.shape[1]
    W_ia = W_i[:atom_dim]
    W_ib = W_i[atom_dim:]
    W_oa = W_o[:atom_dim]
    W_ob = W_o[atom_dim:]
    W1a = W1[:HIDDEN]
    W1b = jnp.zeros((MOL_PAD, ffn_hidden), f32).at[:feat_dim].set(W1[HIDDEN:])
    mfp = jnp.zeros((MOL_PAD, MOL_PAD), f32).at[:n_mol, :feat_dim].set(
        molecule_features)
    W2p = jnp.zeros((ffn_hidden, HIDDEN), f32).at[:, :1].set(W2)
    b22 = jnp.zeros((1, HIDDEN), f32).at[0, 0].set(b2[0])
    b_o2 = b_o.reshape(1, HIDDEN)
    b12 = b1.reshape(1, ffn_hidden)

    afW = _mm_small(af_pad, W_ia)
    g0 = _gather_rows(afW, srcr)
    hr = _h0_kernel(g0, bfr, W_ib)
    h0r = hr

    for _ in range(DEPTH - 1):
        partials = _segment_sum(hr, dstr, zeros_atoms)
        amW = _mm_partials(partials, W_h)
        g = _gather_rows(amW, srcr)
        hr = _update_kernel(h0r, g, hr, W_h)

    partials = _segment_sum(hr, dstr, zeros_atoms)
    out = _final_kernel(af_pad, partials, a2m3, W_oa, W_ob, b_o2,
                        mfp, W1a, W1b, b12, W2p, b22, ffn_hidden)
    return out[:n_mol, :1]


# speedup vs baseline: 2.5218x; 2.4694x over previous
import functools

import jax
import jax.numpy as jnp
from jax import lax
from jax.experimental import pallas as pl
from jax.experimental.pallas import tpu as pltpu
from jax.experimental.pallas import tpu_sc as plsc

N_ATOMS = 10000
N_BONDS = 320000
HIDDEN = 128
DEPTH = 3

NC = 2
NS = 16
NW = NC * NS
EPW = N_BONDS // NW
CHUNK = 80
NCHUNK = EPW // CHUNK
A_PAD = 10240

ATOM_BLK = 1024
EDGE_BLK = 2560
MOL_PAD = 256

assert N_BONDS % EDGE_BLK == 0 and A_PAD % ATOM_BLK == 0
assert EPW == NCHUNK * CHUNK and CHUNK % 8 == 0 and CHUNK <= 128


def _sc_mesh():
    return plsc.VectorSubcoreMesh(
        core_axis_name="c", subcore_axis_name="s",
        num_cores=NC, num_subcores=NS)


def _gather_body(table_hbm, idx_hbm, out_hbm, idx_v, rows_v, sem):
    c = lax.axis_index("c")
    s = lax.axis_index("s")
    w = c * NS + s
    pltpu.sync_copy(idx_hbm.at[w], idx_v)

    def body(j, carry):
        pltpu.async_copy(table_hbm.at[idx_v.at[j]], rows_v, sem).wait()
        pltpu.sync_copy(rows_v, out_hbm.at[pl.ds(w * EPW + j * CHUNK, CHUNK)])
        return carry

    lax.fori_loop(0, NCHUNK, body, 0)


@functools.cache
def _build_gather():
    return pl.kernel(
        _gather_body,
        out_type=jax.ShapeDtypeStruct((N_BONDS, HIDDEN), jnp.float32),
        mesh=_sc_mesh(),
        scratch_types=[
            pltpu.VMEM((NCHUNK, CHUNK), jnp.int32),
            pltpu.VMEM((CHUNK, HIDDEN), jnp.float32),
            pltpu.SemaphoreType.DMA,
        ],
    )


def _gather_rows(table, idx3):
    return _build_gather()(table, idx3)


def _segsum_body(h_hbm, idx_hbm, zeros_hbm, out_hbm, idx_v, rows_v, acc_sh):
    c = lax.axis_index("c")
    s = lax.axis_index("s")
    w = c * NS + s
    rows_per_tile = A_PAD // NS
    pltpu.sync_copy(zeros_hbm.at[pl.ds(s * rows_per_tile, rows_per_tile)],
                    acc_sh.at[pl.ds(s * rows_per_tile, rows_per_tile)])
    pltpu.sync_copy(idx_hbm.at[w], idx_v)
    plsc.subcore_barrier()

    def body(j, carry):
        pltpu.sync_copy(h_hbm.at[pl.ds(w * EPW + j * CHUNK, CHUNK)], rows_v)
        pltpu.sync_copy(rows_v, acc_sh.at[idx_v.at[j]], add=True)
        return carry

    lax.fori_loop(0, NCHUNK, body, 0)
    plsc.subcore_barrier()
    pltpu.sync_copy(acc_sh.at[pl.ds(s * rows_per_tile, rows_per_tile)],
                    out_hbm.at[c, pl.ds(s * rows_per_tile, rows_per_tile)])


@functools.cache
def _build_segsum():
    return pl.kernel(
        _segsum_body,
        out_type=jax.ShapeDtypeStruct((NC, A_PAD, HIDDEN), jnp.float32),
        mesh=_sc_mesh(),
        scratch_types=[
            pltpu.VMEM((NCHUNK, CHUNK), jnp.int32),
            pltpu.VMEM((CHUNK, HIDDEN), jnp.float32),
            pltpu.VMEM_SHARED((A_PAD, HIDDEN), jnp.float32),
        ],
    )


def _segment_sum(h, idx3, zeros):
    return _build_segsum()(h, idx3, zeros)


def _lane_swap_body(shift, x_ref, o_ref):
    x = x_ref[...]
    fwd = pltpu.roll(x, 128 - shift, axis=1)
    bwd = pltpu.roll(x, shift, axis=1)
    lane = lax.broadcasted_iota(jnp.int32, x.shape, 1)
    even = (lane // shift) % 2 == 0
    o_ref[...] = jnp.where(even, fwd, bwd)


def _lane_swap(x, shift):
    rows = x.shape[0]
    blk = 1000
    return pl.pallas_call(
        functools.partial(_lane_swap_body, shift),
        grid=(rows // blk,),
        in_specs=[pl.BlockSpec((blk, 128), lambda i: (i, 0))],
        out_specs=pl.BlockSpec((blk, 128), lambda i: (i, 0)),
        out_shape=jax.ShapeDtypeStruct(x.shape, x.dtype),
    )(x)


def _mm_small_body(x_ref, w_ref, o_ref):
    o_ref[...] = jnp.dot(x_ref[...], w_ref[...],
                         preferred_element_type=jnp.float32)


def _mm_small(x, w):
    grid = A_PAD // ATOM_BLK
    return pl.pallas_call(
        _mm_small_body,
        grid=(grid,),
        in_specs=[pl.BlockSpec((ATOM_BLK, HIDDEN), lambda i: (i, 0)),
                  pl.BlockSpec((HIDDEN, HIDDEN), lambda i: (0, 0))],
        out_specs=pl.BlockSpec((ATOM_BLK, HIDDEN), lambda i: (i, 0)),
        out_shape=jax.ShapeDtypeStruct((A_PAD, HIDDEN), jnp.float32),
    )(x, w)


def _mm_partial_body(p_ref, w_ref, o_ref):
    x = p_ref[0] + p_ref[1]
    o_ref[...] = jnp.dot(x, w_ref[...], preferred_element_type=jnp.float32)


def _mm_partials(p, w):
    grid = A_PAD // ATOM_BLK
    return pl.pallas_call(
        _mm_partial_body,
        grid=(grid,),
        in_specs=[pl.BlockSpec((NC, ATOM_BLK, HIDDEN), lambda i: (0, i, 0)),
                  pl.BlockSpec((HIDDEN, HIDDEN), lambda i: (0, 0))],
        out_specs=pl.BlockSpec((ATOM_BLK, HIDDEN), lambda i: (i, 0)),
        out_shape=jax.ShapeDtypeStruct((A_PAD, HIDDEN), jnp.float32),
    )(p, w)


def _h0_body(g_ref, bf_ref, w_ref, o_ref):
    acc = g_ref[...] + jnp.dot(bf_ref[...], w_ref[...],
                               preferred_element_type=jnp.float32)
    o_ref[...] = jnp.maximum(acc, 0.0)


def _h0_kernel(g0, bfr, w_ib):
    grid = N_BONDS // EDGE_BLK
    bd = bfr.shape[1]
    return pl.pallas_call(
        _h0_body,
        grid=(grid,),
        in_specs=[pl.BlockSpec((EDGE_BLK, HIDDEN), lambda i: (i, 0)),
                  pl.BlockSpec((EDGE_BLK, bd), lambda i: (i, 0)),
                  pl.BlockSpec((bd, HIDDEN), lambda i: (0, 0))],
        out_specs=pl.BlockSpec((EDGE_BLK, HIDDEN), lambda i: (i, 0)),
        out_shape=jax.ShapeDtypeStruct((N_BONDS, HIDDEN), jnp.float32),
    )(g0, bfr, w_ib)


def _update_body(h0_ref, g_ref, h_ref, w_ref, o_ref):
    hw = jnp.dot(h_ref[...], w_ref[...], preferred_element_type=jnp.float32)
    o_ref[...] = jnp.maximum(h0_ref[...] + g_ref[...] - hw, 0.0)


def _update_kernel(h0r, g, hr, w_h):
    grid = N_BONDS // EDGE_BLK
    spec = pl.BlockSpec((EDGE_BLK, HIDDEN), lambda i: (i, 0))
    return pl.pallas_call(
        _update_body,
        grid=(grid,),
        in_specs=[spec, spec, spec,
                  pl.BlockSpec((HIDDEN, HIDDEN), lambda i: (0, 0))],
        out_specs=pl.BlockSpec((EDGE_BLK, HIDDEN), lambda i: (i, 0)),
        out_shape=jax.ShapeDtypeStruct((N_BONDS, HIDDEN), jnp.float32),
    )(h0r, g, hr, w_h)


def _final_body(af_ref, p_ref, a2m_ref, woa_ref, wob_ref, bo_ref,
                mf_ref, w1a_ref, w1b_ref, b1_ref, w2_ref, b2_ref,
                o_ref, msum_ref, cnt_ref):
    i = pl.program_id(0)

    @pl.when(i == 0)
    def _():
        msum_ref[...] = jnp.zeros_like(msum_ref)
        cnt_ref[...] = jnp.zeros_like(cnt_ref)

    m = p_ref[0] + p_ref[1]
    ah = jnp.dot(af_ref[...], woa_ref[...], preferred_element_type=jnp.float32)
    ah = ah + jnp.dot(m, wob_ref[...], preferred_element_type=jnp.float32)
    ah = jnp.maximum(ah + bo_ref[...], 0.0)
    ids = a2m_ref[0]
    onehot = (lax.broadcasted_iota(jnp.int32, (MOL_PAD, ATOM_BLK), 0)
              == ids).astype(jnp.float32)
    msum_ref[...] += jnp.dot(onehot, ah, preferred_element_type=jnp.float32)
    cnt_ref[...] += jnp.dot(onehot,
                            jnp.ones((ATOM_BLK, HIDDEN), jnp.float32),
                            preferred_element_type=jnp.float32)

    @pl.when(i == pl.num_programs(0) - 1)
    def _():
        mol = msum_ref[...] / jnp.maximum(cnt_ref[...], 1.0)
        x = jnp.dot(mol, w1a_ref[...], preferred_element_type=jnp.float32)
        x = x + jnp.dot(mf_ref[...], w1b_ref[...],
                        preferred_element_type=jnp.float32)
        x = jax.nn.sigmoid(x + b1_ref[...])
        o = jnp.dot(x, w2_ref[...], preferred_element_type=jnp.float32)
        o_ref[...] = jax.nn.sigmoid(o + b2_ref[...])


def _final_kernel(af, partials, a2m3, w_oa, w_ob, b_o2, mfp, w1a, w1bp,
                  b12, w2p, b22, ffn_hidden):
    grid = A_PAD // ATOM_BLK
    const = lambda shape: pl.BlockSpec(shape, lambda i: tuple(0 for _ in shape))
    return pl.pallas_call(
        _final_body,
        grid=(grid,),
        in_specs=[
            pl.BlockSpec((ATOM_BLK, HIDDEN), lambda i: (i, 0)),
            pl.BlockSpec((NC, ATOM_BLK, HIDDEN), lambda i: (0, i, 0)),
            pl.BlockSpec((1, 1, ATOM_BLK), lambda i: (i, 0, 0)),
            const((HIDDEN, HIDDEN)),
            const((HIDDEN, HIDDEN)),
            const((1, HIDDEN)),
            const((MOL_PAD, MOL_PAD)),
            const((HIDDEN, ffn_hidden)),
            const((MOL_PAD, ffn_hidden)),
            const((1, ffn_hidden)),
            const((ffn_hidden, HIDDEN)),
            const((1, HIDDEN)),
        ],
        out_specs=pl.BlockSpec((MOL_PAD, HIDDEN), lambda i: (0, 0)),
        out_shape=jax.ShapeDtypeStruct((MOL_PAD, HIDDEN), jnp.float32),
        scratch_shapes=[pltpu.VMEM((MOL_PAD, HIDDEN), jnp.float32),
                        pltpu.VMEM((MOL_PAD, HIDDEN), jnp.float32)],
    )(af, partials, a2m3, w_oa, w_ob, b_o2, mfp, w1a, w1bp, b12, w2p, b22)


def kernel(atom_features, bond_features, bond_index, molecule_features,
           atom_to_molecule, W_i, W_h, W_o, b_o, W1, b1, W2, b2):
    f32 = jnp.float32
    n_mol, feat_dim = molecule_features.shape
    ffn_hidden = W1.shape[1]

    bi = bond_index.astype(jnp.int32).reshape(2 * N_BONDS // 128, 128)
    bir = _lane_swap(bi, 1)
    srcr = bir[:N_BONDS // 128].reshape(NW, NCHUNK, CHUNK)
    dstr = bir[N_BONDS // 128:].reshape(NW, NCHUNK, CHUNK)
    bd = bond_features.shape[1]
    bfr = _lane_swap(
        bond_features.reshape(N_BONDS * bd // 128, 128), bd
    ).reshape(N_BONDS, bd)
    a2m_pad = jnp.full((A_PAD,), MOL_PAD - 1, jnp.int32).at[:N_ATOMS].set(
        atom_to_molecule.astype(jnp.int32))
    a2m3 = a2m_pad.reshape(A_PAD // ATOM_BLK, 1, ATOM_BLK)
    zeros_atoms = jnp.zeros((A_PAD, HIDDEN), f32)
    af_pad = jnp.zeros((A_PAD, atom_features.shape[1]), f32).at[:N_ATOMS].set(
        atom_features)

    atom_dim = atom_features.shape[1]
    W_ia = W_i[:atom_dim]
    W_ib = W_i[atom_dim:]
    W_oa = W_o[:atom_dim]
    W_ob = W_o[atom_dim:]
    W1a = W1[:HIDDEN]
    W1b = jnp.zeros((MOL_PAD, ffn_hidden), f32).at[:feat_dim].set(W1[HIDDEN:])
    mfp = jnp.zeros((MOL_PAD, MOL_PAD), f32).at[:n_mol, :feat_dim].set(
        molecule_features)
    W2p = jnp.zeros((ffn_hidden, HIDDEN), f32).at[:, :1].set(W2)
    b22 = jnp.zeros((1, HIDDEN), f32).at[0, 0].set(b2[0])
    b_o2 = b_o.reshape(1, HIDDEN)
    b12 = b1.reshape(1, ffn_hidden)

    afW = _mm_small(af_pad, W_ia)
    g0 = _gather_rows(afW, srcr)
    hr = _h0_kernel(g0, bfr, W_ib)
    h0r = hr

    for _ in range(DEPTH - 1):
        partials = _segment_sum(hr, dstr, zeros_atoms)
        amW = _mm_partials(partials, W_h)
        g = _gather_rows(amW, srcr)
        hr = _update_kernel(h0r, g, hr, W_h)

    partials = _segment_sum(hr, dstr, zeros_atoms)
    out = _final_kernel(af_pad, partials, a2m3, W_oa, W_ob, b_o2,
                        mfp, W1a, W1b, b12, W2p, b22, ffn_hidden)
    return out[:n_mol, :1]


# speedup vs baseline: 2.9699x; 1.1777x over previous
import functools

import jax
import jax.numpy as jnp
from jax import lax
from jax.experimental import pallas as pl
from jax.experimental.pallas import tpu as pltpu
from jax.experimental.pallas import tpu_sc as plsc

N_ATOMS = 10000
N_BONDS = 320000
HIDDEN = 128
DEPTH = 3

NC = 2
NS = 16
NW = NC * NS
EPW = N_BONDS // NW
GCH = 80
GNCH = EPW // GCH
GNB = 5
SCH = 40
SNCH = EPW // SCH
SNB = 2
A_PAD = 10240

ATOM_BLK = 1024
EDGE_BLK = 2560
MOL_PAD = 256

assert N_BONDS % EDGE_BLK == 0 and A_PAD % ATOM_BLK == 0
assert EPW == GNCH * GCH and GCH % 8 == 0 and GCH <= 128 and GNCH % GNB == 0
assert EPW == SNCH * SCH and SCH % 8 == 0 and SCH <= 128 and SNCH % SNB == 0


def _sc_mesh():
    return plsc.VectorSubcoreMesh(
        core_axis_name="c", subcore_axis_name="s",
        num_cores=NC, num_subcores=NS)


def _gather_body(table_hbm, idx_hbm, out_hbm, idx_v, bufs, sems):
    c = lax.axis_index("c")
    s = lax.axis_index("s")
    w = c * NS + s
    pltpu.sync_copy(idx_hbm.at[w], idx_v)

    def body(i, carry):
        j0 = i * GNB
        for k in range(GNB):
            pltpu.async_copy(table_hbm.at[idx_v.at[j0 + k]], bufs.at[k],
                             sems[k])
        for k in range(GNB):
            pltpu.make_async_copy(table_hbm.at[idx_v.at[j0 + k]], bufs.at[k],
                                  sems[k]).wait()
            pltpu.sync_copy(
                bufs.at[k],
                out_hbm.at[pl.ds(w * EPW + (j0 + k) * GCH, GCH)])
        return carry

    lax.fori_loop(0, GNCH // GNB, body, 0)


@functools.cache
def _build_gather():
    return pl.kernel(
        _gather_body,
        out_type=jax.ShapeDtypeStruct((N_BONDS, HIDDEN), jnp.float32),
        mesh=_sc_mesh(),
        scratch_types=[
            pltpu.VMEM((GNCH, GCH), jnp.int32),
            pltpu.VMEM((GNB, GCH, HIDDEN), jnp.float32),
            [pltpu.SemaphoreType.DMA] * GNB,
        ],
    )


def _gather_rows(table, idx3):
    return _build_gather()(table, idx3)


def _segsum_body(h_hbm, idx_hbm, zeros_hbm, out_hbm, idx_v, rows_v, acc_sh,
                 sems):
    c = lax.axis_index("c")
    s = lax.axis_index("s")
    w = c * NS + s
    rows_per_tile = A_PAD // NS
    pltpu.sync_copy(zeros_hbm.at[pl.ds(s * rows_per_tile, rows_per_tile)],
                    acc_sh.at[pl.ds(s * rows_per_tile, rows_per_tile)])
    pltpu.sync_copy(idx_hbm.at[w], idx_v)
    plsc.subcore_barrier()

    def body(i, carry):
        j0 = i * SNB
        for k in range(SNB):
            pltpu.async_copy(
                h_hbm.at[pl.ds(w * EPW + (j0 + k) * SCH, SCH)],
                rows_v.at[k], sems[k])
        for k in range(SNB):
            pltpu.make_async_copy(
                h_hbm.at[pl.ds(w * EPW + (j0 + k) * SCH, SCH)],
                rows_v.at[k], sems[k]).wait()
            pltpu.sync_copy(rows_v.at[k], acc_sh.at[idx_v.at[j0 + k]],
                            add=True)
        return carry

    lax.fori_loop(0, SNCH // SNB, body, 0)
    plsc.subcore_barrier()
    pltpu.sync_copy(acc_sh.at[pl.ds(s * rows_per_tile, rows_per_tile)],
                    out_hbm.at[c, pl.ds(s * rows_per_tile, rows_per_tile)])


@functools.cache
def _build_segsum():
    return pl.kernel(
        _segsum_body,
        out_type=jax.ShapeDtypeStruct((NC, A_PAD, HIDDEN), jnp.float32),
        mesh=_sc_mesh(),
        scratch_types=[
            pltpu.VMEM((SNCH, SCH), jnp.int32),
            pltpu.VMEM((SNB, SCH, HIDDEN), jnp.float32),
            pltpu.VMEM_SHARED((A_PAD, HIDDEN), jnp.float32),
            [pltpu.SemaphoreType.DMA] * SNB,
        ],
    )


def _segment_sum(h, idx3, zeros):
    return _build_segsum()(h, idx3, zeros)


def _lane_swap_body(shift, x_ref, o_ref):
    x = x_ref[...]
    fwd = pltpu.roll(x, 128 - shift, axis=1)
    bwd = pltpu.roll(x, shift, axis=1)
    lane = lax.broadcasted_iota(jnp.int32, x.shape, 1)
    even = (lane // shift) % 2 == 0
    o_ref[...] = jnp.where(even, fwd, bwd)


def _lane_swap(x, shift):
    rows = x.shape[0]
    blk = 1000
    return pl.pallas_call(
        functools.partial(_lane_swap_body, shift),
        grid=(rows // blk,),
        in_specs=[pl.BlockSpec((blk, 128), lambda i: (i, 0))],
        out_specs=pl.BlockSpec((blk, 128), lambda i: (i, 0)),
        out_shape=jax.ShapeDtypeStruct(x.shape, x.dtype),
    )(x)


def _mm_small_body(x_ref, w_ref, o_ref):
    o_ref[...] = jnp.dot(x_ref[...], w_ref[...],
                         preferred_element_type=jnp.float32)


def _mm_small(x, w):
    grid = A_PAD // ATOM_BLK
    return pl.pallas_call(
        _mm_small_body,
        grid=(grid,),
        in_specs=[pl.BlockSpec((ATOM_BLK, HIDDEN), lambda i: (i, 0)),
                  pl.BlockSpec((HIDDEN, HIDDEN), lambda i: (0, 0))],
        out_specs=pl.BlockSpec((ATOM_BLK, HIDDEN), lambda i: (i, 0)),
        out_shape=jax.ShapeDtypeStruct((A_PAD, HIDDEN), jnp.float32),
    )(x, w)


def _mm_partial_body(p_ref, w_ref, o_ref):
    x = p_ref[0] + p_ref[1]
    o_ref[...] = jnp.dot(x, w_ref[...], preferred_element_type=jnp.float32)


def _mm_partials(p, w):
    grid = A_PAD // ATOM_BLK
    return pl.pallas_call(
        _mm_partial_body,
        grid=(grid,),
        in_specs=[pl.BlockSpec((NC, ATOM_BLK, HIDDEN), lambda i: (0, i, 0)),
                  pl.BlockSpec((HIDDEN, HIDDEN), lambda i: (0, 0))],
        out_specs=pl.BlockSpec((ATOM_BLK, HIDDEN), lambda i: (i, 0)),
        out_shape=jax.ShapeDtypeStruct((A_PAD, HIDDEN), jnp.float32),
    )(p, w)


def _h0_body(g_ref, bf_ref, w_ref, o_ref):
    x = bf_ref[...]
    up = pltpu.roll(x, EDGE_BLK - 1, axis=0)
    dn = pltpu.roll(x, 1, axis=0)
    row = lax.broadcasted_iota(jnp.int32, x.shape, 0)
    xs = jnp.where(row % 2 == 0, up, dn)
    acc = g_ref[...] + jnp.dot(xs, w_ref[...],
                               preferred_element_type=jnp.float32)
    o_ref[...] = jnp.maximum(acc, 0.0)


def _h0_kernel(g0, bfr, w_ib):
    grid = N_BONDS // EDGE_BLK
    bd = bfr.shape[1]
    return pl.pallas_call(
        _h0_body,
        grid=(grid,),
        in_specs=[pl.BlockSpec((EDGE_BLK, HIDDEN), lambda i: (i, 0)),
                  pl.BlockSpec((EDGE_BLK, bd), lambda i: (i, 0)),
                  pl.BlockSpec((bd, HIDDEN), lambda i: (0, 0))],
        out_specs=pl.BlockSpec((EDGE_BLK, HIDDEN), lambda i: (i, 0)),
        out_shape=jax.ShapeDtypeStruct((N_BONDS, HIDDEN), jnp.float32),
    )(g0, bfr, w_ib)


def _update_body(h0_ref, g_ref, h_ref, w_ref, o_ref):
    hw = jnp.dot(h_ref[...], w_ref[...], preferred_element_type=jnp.float32)
    o_ref[...] = jnp.maximum(h0_ref[...] + g_ref[...] - hw, 0.0)


def _update_kernel(h0r, g, hr, w_h):
    grid = N_BONDS // EDGE_BLK
    spec = pl.BlockSpec((EDGE_BLK, HIDDEN), lambda i: (i, 0))
    return pl.pallas_call(
        _update_body,
        grid=(grid,),
        in_specs=[spec, spec, spec,
                  pl.BlockSpec((HIDDEN, HIDDEN), lambda i: (0, 0))],
        out_specs=pl.BlockSpec((EDGE_BLK, HIDDEN), lambda i: (i, 0)),
        out_shape=jax.ShapeDtypeStruct((N_BONDS, HIDDEN), jnp.float32),
    )(h0r, g, hr, w_h)


def _final_body(af_ref, p_ref, a2m_ref, woa_ref, wob_ref, bo_ref,
                mf_ref, w1a_ref, w1b_ref, b1_ref, w2_ref, b2_ref,
                o_ref, msum_ref, cnt_ref):
    i = pl.program_id(0)

    @pl.when(i == 0)
    def _():
        msum_ref[...] = jnp.zeros_like(msum_ref)
        cnt_ref[...] = jnp.zeros_like(cnt_ref)

    m = p_ref[0] + p_ref[1]
    ah = jnp.dot(af_ref[...], woa_ref[...], preferred_element_type=jnp.float32)
    ah = ah + jnp.dot(m, wob_ref[...], preferred_element_type=jnp.float32)
    ah = jnp.maximum(ah + bo_ref[...], 0.0)
    ids = a2m_ref[0]
    onehot = (lax.broadcasted_iota(jnp.int32, (MOL_PAD, ATOM_BLK), 0)
              == ids).astype(jnp.float32)
    msum_ref[...] += jnp.dot(onehot, ah, preferred_element_type=jnp.float32)
    cnt_ref[...] += jnp.dot(onehot,
                            jnp.ones((ATOM_BLK, HIDDEN), jnp.float32),
                            preferred_element_type=jnp.float32)

    @pl.when(i == pl.num_programs(0) - 1)
    def _():
        mol = msum_ref[...] / jnp.maximum(cnt_ref[...], 1.0)
        x = jnp.dot(mol, w1a_ref[...], preferred_element_type=jnp.float32)
        x = x + jnp.dot(mf_ref[...], w1b_ref[...],
                        preferred_element_type=jnp.float32)
        x = jax.nn.sigmoid(x + b1_ref[...])
        o = jnp.dot(x, w2_ref[...], preferred_element_type=jnp.float32)
        o_ref[...] = jax.nn.sigmoid(o + b2_ref[...])


def _final_kernel(af, partials, a2m3, w_oa, w_ob, b_o2, mfp, w1a, w1bp,
                  b12, w2p, b22, ffn_hidden):
    grid = A_PAD // ATOM_BLK
    const = lambda shape: pl.BlockSpec(shape, lambda i: tuple(0 for _ in shape))
    return pl.pallas_call(
        _final_body,
        grid=(grid,),
        in_specs=[
            pl.BlockSpec((ATOM_BLK, HIDDEN), lambda i: (i, 0)),
            pl.BlockSpec((NC, ATOM_BLK, HIDDEN), lambda i: (0, i, 0)),
            pl.BlockSpec((1, 1, ATOM_BLK), lambda i: (i, 0, 0)),
            const((HIDDEN, HIDDEN)),
            const((HIDDEN, HIDDEN)),
            const((1, HIDDEN)),
            const((MOL_PAD, MOL_PAD)),
            const((HIDDEN, ffn_hidden)),
            const((MOL_PAD, ffn_hidden)),
            const((1, ffn_hidden)),
            const((ffn_hidden, HIDDEN)),
            const((1, HIDDEN)),
        ],
        out_specs=pl.BlockSpec((MOL_PAD, HIDDEN), lambda i: (0, 0)),
        out_shape=jax.ShapeDtypeStruct((MOL_PAD, HIDDEN), jnp.float32),
        scratch_shapes=[pltpu.VMEM((MOL_PAD, HIDDEN), jnp.float32),
                        pltpu.VMEM((MOL_PAD, HIDDEN), jnp.float32)],
    )(af, partials, a2m3, w_oa, w_ob, b_o2, mfp, w1a, w1bp, b12, w2p, b22)


def kernel(atom_features, bond_features, bond_index, molecule_features,
           atom_to_molecule, W_i, W_h, W_o, b_o, W1, b1, W2, b2):
    f32 = jnp.float32
    n_mol, feat_dim = molecule_features.shape
    ffn_hidden = W1.shape[1]

    bi = bond_index.astype(jnp.int32).reshape(2 * N_BONDS // 128, 128)
    bir = _lane_swap(bi, 1)
    srcr = bir[:N_BONDS // 128].reshape(NW, GNCH, GCH)
    dstr = bir[N_BONDS // 128:].reshape(NW, SNCH, SCH)

    a2m_pad = jnp.full((A_PAD,), MOL_PAD - 1, jnp.int32).at[:N_ATOMS].set(
        atom_to_molecule.astype(jnp.int32))
    a2m3 = a2m_pad.reshape(A_PAD // ATOM_BLK, 1, ATOM_BLK)
    zeros_atoms = jnp.zeros((A_PAD, HIDDEN), f32)
    af_pad = jnp.zeros((A_PAD, atom_features.shape[1]), f32).at[:N_ATOMS].set(
        atom_features)

    atom_dim = atom_features.shape[1]
    W_ia = W_i[:atom_dim]
    W_ib = W_i[atom_dim:]
    W_oa = W_o[:atom_dim]
    W_ob = W_o[atom_dim:]
    W1a = W1[:HIDDEN]
    W1b = jnp.zeros((MOL_PAD, ffn_hidden), f32).at[:feat_dim].set(W1[HIDDEN:])
    mfp = jnp.zeros((MOL_PAD, MOL_PAD), f32).at[:n_mol, :feat_dim].set(
        molecule_features)
    W2p = jnp.zeros((ffn_hidden, HIDDEN), f32).at[:, :1].set(W2)
    b22 = jnp.zeros((1, HIDDEN), f32).at[0, 0].set(b2[0])
    b_o2 = b_o.reshape(1, HIDDEN)
    b12 = b1.reshape(1, ffn_hidden)

    afW = _mm_small(af_pad, W_ia)
    g0 = _gather_rows(afW, srcr)
    hr = _h0_kernel(g0, bond_features, W_ib)
    h0r = hr

    for _ in range(DEPTH - 1):
        partials = _segment_sum(hr, dstr, zeros_atoms)
        amW = _mm_partials(partials, W_h)
        g = _gather_rows(amW, srcr)
        hr = _update_kernel(h0r, g, hr, W_h)

    partials = _segment_sum(hr, dstr, zeros_atoms)
    out = _final_kernel(af_pad, partials, a2m3, W_oa, W_ob, b_o2,
                        mfp, W1a, W1b, b12, W2p, b22, ffn_hidden)
    return out[:n_mol, :1]


# speedup vs baseline: 3.1031x; 1.0449x over previous
import functools

import jax
import jax.numpy as jnp
from jax import lax
from jax.experimental import pallas as pl
from jax.experimental.pallas import tpu as pltpu
from jax.experimental.pallas import tpu_sc as plsc

N_ATOMS = 10000
N_BONDS = 320000
HIDDEN = 128
DEPTH = 3

NC = 2
NS = 16
NW = NC * NS
EPW = N_BONDS // NW
GCH = 80
GNCH = EPW // GCH
GNB = 5
SCH = 40
SNCH = EPW // SCH
SNB = 2
A_PAD = 10240

ATOM_BLK = 1024
EDGE_BLK = 2560
MOL_PAD = 256

assert N_BONDS % EDGE_BLK == 0 and A_PAD % ATOM_BLK == 0
assert EPW == GNCH * GCH and GCH % 8 == 0 and GCH <= 128 and GNCH % GNB == 0
assert EPW == SNCH * SCH and SCH % 8 == 0 and SCH <= 128 and SNCH % SNB == 0


def _sc_mesh():
    return plsc.VectorSubcoreMesh(
        core_axis_name="c", subcore_axis_name="s",
        num_cores=NC, num_subcores=NS)


def _gather_body(table_hbm, idx_hbm, out_hbm, idx_v, bufs, sems):
    c = lax.axis_index("c")
    s = lax.axis_index("s")
    w = c * NS + s
    pltpu.sync_copy(idx_hbm.at[w], idx_v)

    def body(i, carry):
        j0 = i * GNB
        for k in range(GNB):
            pltpu.async_copy(table_hbm.at[idx_v.at[j0 + k]], bufs.at[k],
                             sems[k])
        for k in range(GNB):
            pltpu.make_async_copy(table_hbm.at[idx_v.at[j0 + k]], bufs.at[k],
                                  sems[k]).wait()
            pltpu.sync_copy(
                bufs.at[k],
                out_hbm.at[pl.ds(w * EPW + (j0 + k) * GCH, GCH)])
        return carry

    lax.fori_loop(0, GNCH // GNB, body, 0)


@functools.cache
def _build_gather():
    return pl.kernel(
        _gather_body,
        out_type=jax.ShapeDtypeStruct((N_BONDS, HIDDEN), jnp.float32),
        mesh=_sc_mesh(),
        scratch_types=[
            pltpu.VMEM((GNCH, GCH), jnp.int32),
            pltpu.VMEM((GNB, GCH, HIDDEN), jnp.float32),
            [pltpu.SemaphoreType.DMA] * GNB,
        ],
    )


def _gather_rows(table, idx3):
    return _build_gather()(table, idx3)


def _segsum_body(h_hbm, idx_hbm, zeros_hbm, out_hbm, idx_v, rows_v, acc_sh,
                 sems):
    c = lax.axis_index("c")
    s = lax.axis_index("s")
    w = c * NS + s
    rows_per_tile = A_PAD // NS
    pltpu.sync_copy(zeros_hbm.at[pl.ds(s * rows_per_tile, rows_per_tile)],
                    acc_sh.at[pl.ds(s * rows_per_tile, rows_per_tile)])
    pltpu.sync_copy(idx_hbm.at[w], idx_v)
    plsc.subcore_barrier()

    def body(i, carry):
        j0 = i * SNB
        for k in range(SNB):
            pltpu.async_copy(
                h_hbm.at[pl.ds(w * EPW + (j0 + k) * SCH, SCH)],
                rows_v.at[k], sems[k])
        for k in range(SNB):
            pltpu.make_async_copy(
                h_hbm.at[pl.ds(w * EPW + (j0 + k) * SCH, SCH)],
                rows_v.at[k], sems[k]).wait()
            pltpu.sync_copy(rows_v.at[k], acc_sh.at[idx_v.at[j0 + k]],
                            add=True)
        return carry

    lax.fori_loop(0, SNCH // SNB, body, 0)
    plsc.subcore_barrier()
    pltpu.sync_copy(acc_sh.at[pl.ds(s * rows_per_tile, rows_per_tile)],
                    out_hbm.at[c, pl.ds(s * rows_per_tile, rows_per_tile)])


@functools.cache
def _build_segsum():
    return pl.kernel(
        _segsum_body,
        out_type=jax.ShapeDtypeStruct((NC, A_PAD, HIDDEN), jnp.float32),
        mesh=_sc_mesh(),
        scratch_types=[
            pltpu.VMEM((SNCH, SCH), jnp.int32),
            pltpu.VMEM((SNB, SCH, HIDDEN), jnp.float32),
            pltpu.VMEM_SHARED((A_PAD, HIDDEN), jnp.float32),
            [pltpu.SemaphoreType.DMA] * SNB,
        ],
    )


def _segment_sum(h, idx3, zeros):
    return _build_segsum()(h, idx3, zeros)


def _lane_swap_body(shift, x_ref, o_ref):
    x = x_ref[...]
    fwd = pltpu.roll(x, 128 - shift, axis=1)
    bwd = pltpu.roll(x, shift, axis=1)
    lane = lax.broadcasted_iota(jnp.int32, x.shape, 1)
    even = (lane // shift) % 2 == 0
    o_ref[...] = jnp.where(even, fwd, bwd)


def _lane_swap(x, shift):
    rows = x.shape[0]
    blk = 1000
    return pl.pallas_call(
        functools.partial(_lane_swap_body, shift),
        grid=(rows // blk,),
        in_specs=[pl.BlockSpec((blk, 128), lambda i: (i, 0))],
        out_specs=pl.BlockSpec((blk, 128), lambda i: (i, 0)),
        out_shape=jax.ShapeDtypeStruct(x.shape, x.dtype),
    )(x)


def _mm_small_body(x_ref, w_ref, o_ref):
    o_ref[...] = jnp.dot(x_ref[...], w_ref[...],
                         preferred_element_type=jnp.float32)


def _mm_small(x, w):
    grid = A_PAD // ATOM_BLK
    return pl.pallas_call(
        _mm_small_body,
        grid=(grid,),
        in_specs=[pl.BlockSpec((ATOM_BLK, HIDDEN), lambda i: (i, 0)),
                  pl.BlockSpec((HIDDEN, HIDDEN), lambda i: (0, 0))],
        out_specs=pl.BlockSpec((ATOM_BLK, HIDDEN), lambda i: (i, 0)),
        out_shape=jax.ShapeDtypeStruct((A_PAD, HIDDEN), jnp.float32),
    )(x, w)


def _mm_partial_body(p_ref, w_ref, o_ref):
    x = p_ref[0] + p_ref[1]
    o_ref[...] = jnp.dot(x, w_ref[...], preferred_element_type=jnp.float32)


def _mm_partials(p, w):
    grid = A_PAD // ATOM_BLK
    return pl.pallas_call(
        _mm_partial_body,
        grid=(grid,),
        in_specs=[pl.BlockSpec((NC, ATOM_BLK, HIDDEN), lambda i: (0, i, 0)),
                  pl.BlockSpec((HIDDEN, HIDDEN), lambda i: (0, 0))],
        out_specs=pl.BlockSpec((ATOM_BLK, HIDDEN), lambda i: (i, 0)),
        out_shape=jax.ShapeDtypeStruct((A_PAD, HIDDEN), jnp.float32),
    )(p, w)


def _h0_body(g_ref, bf_ref, w_ref, o_ref):
    x = bf_ref[...]
    up = pltpu.roll(x, EDGE_BLK - 1, axis=1)
    dn = pltpu.roll(x, 1, axis=1)
    lane = lax.broadcasted_iota(jnp.int32, x.shape, 1)
    xs = jnp.where(lane % 2 == 0, up, dn)
    mm = lax.dot_general(xs, w_ref[...], (((0,), (0,)), ((), ())),
                         preferred_element_type=jnp.float32)
    o_ref[...] = jnp.maximum(g_ref[...] + mm, 0.0)


def _h0_kernel(g0, bfT, w_ib):
    grid = N_BONDS // EDGE_BLK
    bd = bfT.shape[0]
    return pl.pallas_call(
        _h0_body,
        grid=(grid,),
        in_specs=[pl.BlockSpec((EDGE_BLK, HIDDEN), lambda i: (i, 0)),
                  pl.BlockSpec((bd, EDGE_BLK), lambda i: (0, i)),
                  pl.BlockSpec((bd, HIDDEN), lambda i: (0, 0))],
        out_specs=pl.BlockSpec((EDGE_BLK, HIDDEN), lambda i: (i, 0)),
        out_shape=jax.ShapeDtypeStruct((N_BONDS, HIDDEN), jnp.float32),
    )(g0, bfT, w_ib)


def _update_body(h0_ref, g_ref, h_ref, w_ref, o_ref):
    hw = jnp.dot(h_ref[...], w_ref[...], preferred_element_type=jnp.float32)
    o_ref[...] = jnp.maximum(h0_ref[...] + g_ref[...] - hw, 0.0)


def _update_kernel(h0r, g, hr, w_h):
    grid = N_BONDS // EDGE_BLK
    spec = pl.BlockSpec((EDGE_BLK, HIDDEN), lambda i: (i, 0))
    return pl.pallas_call(
        _update_body,
        grid=(grid,),
        in_specs=[spec, spec, spec,
                  pl.BlockSpec((HIDDEN, HIDDEN), lambda i: (0, 0))],
        out_specs=pl.BlockSpec((EDGE_BLK, HIDDEN), lambda i: (i, 0)),
        out_shape=jax.ShapeDtypeStruct((N_BONDS, HIDDEN), jnp.float32),
    )(h0r, g, hr, w_h)


def _final_body(af_ref, p_ref, a2m_ref, woa_ref, wob_ref, bo_ref,
                mf_ref, w1a_ref, w1b_ref, b1_ref, w2_ref, b2_ref,
                o_ref, msum_ref, cnt_ref):
    i = pl.program_id(0)

    @pl.when(i == 0)
    def _():
        msum_ref[...] = jnp.zeros_like(msum_ref)
        cnt_ref[...] = jnp.zeros_like(cnt_ref)

    m = p_ref[0] + p_ref[1]
    ah = jnp.dot(af_ref[...], woa_ref[...], preferred_element_type=jnp.float32)
    ah = ah + jnp.dot(m, wob_ref[...], preferred_element_type=jnp.float32)
    ah = jnp.maximum(ah + bo_ref[...], 0.0)
    ids = a2m_ref[0]
    onehot = (lax.broadcasted_iota(jnp.int32, (MOL_PAD, ATOM_BLK), 0)
              == ids).astype(jnp.float32)
    msum_ref[...] += jnp.dot(onehot, ah, preferred_element_type=jnp.float32)
    cnt_ref[...] += jnp.dot(onehot,
                            jnp.ones((ATOM_BLK, HIDDEN), jnp.float32),
                            preferred_element_type=jnp.float32)

    @pl.when(i == pl.num_programs(0) - 1)
    def _():
        mol = msum_ref[...] / jnp.maximum(cnt_ref[...], 1.0)
        x = jnp.dot(mol, w1a_ref[...], preferred_element_type=jnp.float32)
        x = x + jnp.dot(mf_ref[...], w1b_ref[...],
                        preferred_element_type=jnp.float32)
        x = jax.nn.sigmoid(x + b1_ref[...])
        o = jnp.dot(x, w2_ref[...], preferred_element_type=jnp.float32)
        o_ref[...] = jax.nn.sigmoid(o + b2_ref[...])


def _final_kernel(af, partials, a2m3, w_oa, w_ob, b_o2, mfp, w1a, w1bp,
                  b12, w2p, b22, ffn_hidden):
    grid = A_PAD // ATOM_BLK
    const = lambda shape: pl.BlockSpec(shape, lambda i: tuple(0 for _ in shape))
    return pl.pallas_call(
        _final_body,
        grid=(grid,),
        in_specs=[
            pl.BlockSpec((ATOM_BLK, HIDDEN), lambda i: (i, 0)),
            pl.BlockSpec((NC, ATOM_BLK, HIDDEN), lambda i: (0, i, 0)),
            pl.BlockSpec((1, 1, ATOM_BLK), lambda i: (i, 0, 0)),
            const((HIDDEN, HIDDEN)),
            const((HIDDEN, HIDDEN)),
            const((1, HIDDEN)),
            const((MOL_PAD, MOL_PAD)),
            const((HIDDEN, ffn_hidden)),
            const((MOL_PAD, ffn_hidden)),
            const((1, ffn_hidden)),
            const((ffn_hidden, HIDDEN)),
            const((1, HIDDEN)),
        ],
        out_specs=pl.BlockSpec((MOL_PAD, HIDDEN), lambda i: (0, 0)),
        out_shape=jax.ShapeDtypeStruct((MOL_PAD, HIDDEN), jnp.float32),
        scratch_shapes=[pltpu.VMEM((MOL_PAD, HIDDEN), jnp.float32),
                        pltpu.VMEM((MOL_PAD, HIDDEN), jnp.float32)],
    )(af, partials, a2m3, w_oa, w_ob, b_o2, mfp, w1a, w1bp, b12, w2p, b22)


def kernel(atom_features, bond_features, bond_index, molecule_features,
           atom_to_molecule, W_i, W_h, W_o, b_o, W1, b1, W2, b2):
    f32 = jnp.float32
    n_mol, feat_dim = molecule_features.shape
    ffn_hidden = W1.shape[1]

    bi = bond_index.astype(jnp.int32).reshape(2 * N_BONDS // 128, 128)
    bir = _lane_swap(bi, 1)
    srcr = bir[:N_BONDS // 128].reshape(NW, GNCH, GCH)
    dstr = bir[N_BONDS // 128:].reshape(NW, SNCH, SCH)

    a2m_pad = jnp.full((A_PAD,), MOL_PAD - 1, jnp.int32).at[:N_ATOMS].set(
        atom_to_molecule.astype(jnp.int32))
    a2m3 = a2m_pad.reshape(A_PAD // ATOM_BLK, 1, ATOM_BLK)
    zeros_atoms = jnp.zeros((A_PAD, HIDDEN), f32)
    af_pad = jnp.zeros((A_PAD, atom_features.shape[1]), f32).at[:N_ATOMS].set(
        atom_features)

    atom_dim = atom_features.shape[1]
    W_ia = W_i[:atom_dim]
    W_ib = W_i[atom_dim:]
    W_oa = W_o[:atom_dim]
    W_ob = W_o[atom_dim:]
    W1a = W1[:HIDDEN]
    W1b = jnp.zeros((MOL_PAD, ffn_hidden), f32).at[:feat_dim].set(W1[HIDDEN:])
    mfp = jnp.zeros((MOL_PAD, MOL_PAD), f32).at[:n_mol, :feat_dim].set(
        molecule_features)
    W2p = jnp.zeros((ffn_hidden, HIDDEN), f32).at[:, :1].set(W2)
    b22 = jnp.zeros((1, HIDDEN), f32).at[0, 0].set(b2[0])
    b_o2 = b_o.reshape(1, HIDDEN)
    b12 = b1.reshape(1, ffn_hidden)

    afW = _mm_small(af_pad, W_ia)
    g0 = _gather_rows(afW, srcr)
    hr = _h0_kernel(g0, bond_features.T, W_ib)
    h0r = hr

    for _ in range(DEPTH - 1):
        partials = _segment_sum(hr, dstr, zeros_atoms)
        amW = _mm_partials(partials, W_h)
        g = _gather_rows(amW, srcr)
        hr = _update_kernel(h0r, g, hr, W_h)

    partials = _segment_sum(hr, dstr, zeros_atoms)
    out = _final_kernel(af_pad, partials, a2m3, W_oa, W_ob, b_o2,
                        mfp, W1a, W1b, b12, W2p, b22, ffn_hidden)
    return out[:n_mol, :1]


# speedup vs baseline: 3.2557x; 1.0492x over previous
import functools

import jax
import jax.numpy as jnp
from jax import lax
from jax.experimental import pallas as pl
from jax.experimental.pallas import tpu as pltpu
from jax.experimental.pallas import tpu_sc as plsc

N_ATOMS = 10000
N_BONDS = 320000
HIDDEN = 128
DEPTH = 3

NC = 2
NS = 16
NW = NC * NS
EPW = N_BONDS // NW
GCH = 80
GNCH = EPW // GCH
GNB = 5
SCH = 40
SNCH = EPW // SCH
SNB = 2
A_PAD = 10240

ATOM_BLK = 1024
EDGE_BLK = 6400
MOL_PAD = 256

assert N_BONDS % EDGE_BLK == 0 and A_PAD % ATOM_BLK == 0
assert EPW == GNCH * GCH and GCH % 8 == 0 and GCH <= 128 and GNCH % GNB == 0
assert EPW == SNCH * SCH and SCH % 8 == 0 and SCH <= 128 and SNCH % SNB == 0


def _sc_mesh():
    return plsc.VectorSubcoreMesh(
        core_axis_name="c", subcore_axis_name="s",
        num_cores=NC, num_subcores=NS)


def _gather_body(table_hbm, idx_hbm, out_hbm, idx_v, bufs, sems):
    c = lax.axis_index("c")
    s = lax.axis_index("s")
    w = c * NS + s
    pltpu.sync_copy(idx_hbm.at[w], idx_v)

    def body(i, carry):
        j0 = i * GNB
        for k in range(GNB):
            pltpu.async_copy(table_hbm.at[idx_v.at[j0 + k]], bufs.at[k],
                             sems[k])
        for k in range(GNB):
            pltpu.make_async_copy(table_hbm.at[idx_v.at[j0 + k]], bufs.at[k],
                                  sems[k]).wait()
            pltpu.sync_copy(
                bufs.at[k],
                out_hbm.at[pl.ds(w * EPW + (j0 + k) * GCH, GCH)])
        return carry

    lax.fori_loop(0, GNCH // GNB, body, 0)


@functools.cache
def _build_gather():
    return pl.kernel(
        _gather_body,
        out_type=jax.ShapeDtypeStruct((N_BONDS, HIDDEN), jnp.float32),
        mesh=_sc_mesh(),
        scratch_types=[
            pltpu.VMEM((GNCH, GCH), jnp.int32),
            pltpu.VMEM((GNB, GCH, HIDDEN), jnp.float32),
            [pltpu.SemaphoreType.DMA] * GNB,
        ],
    )


def _gather_rows(table, idx3):
    return _build_gather()(table, idx3)


def _segsum_body(h_hbm, idx_hbm, zeros_hbm, out_hbm, idx_v, rows_v, acc_sh,
                 sems):
    c = lax.axis_index("c")
    s = lax.axis_index("s")
    w = c * NS + s
    rows_per_tile = A_PAD // NS
    pltpu.sync_copy(zeros_hbm.at[pl.ds(s * rows_per_tile, rows_per_tile)],
                    acc_sh.at[pl.ds(s * rows_per_tile, rows_per_tile)])
    pltpu.sync_copy(idx_hbm.at[w], idx_v)
    plsc.subcore_barrier()

    def body(i, carry):
        j0 = i * SNB
        for k in range(SNB):
            pltpu.async_copy(
                h_hbm.at[pl.ds(w * EPW + (j0 + k) * SCH, SCH)],
                rows_v.at[k], sems[k])
        for k in range(SNB):
            pltpu.make_async_copy(
                h_hbm.at[pl.ds(w * EPW + (j0 + k) * SCH, SCH)],
                rows_v.at[k], sems[k]).wait()
            pltpu.sync_copy(rows_v.at[k], acc_sh.at[idx_v.at[j0 + k]],
                            add=True)
        return carry

    lax.fori_loop(0, SNCH // SNB, body, 0)
    plsc.subcore_barrier()
    pltpu.sync_copy(acc_sh.at[pl.ds(s * rows_per_tile, rows_per_tile)],
                    out_hbm.at[c, pl.ds(s * rows_per_tile, rows_per_tile)])


@functools.cache
def _build_segsum():
    return pl.kernel(
        _segsum_body,
        out_type=jax.ShapeDtypeStruct((NC, A_PAD, HIDDEN), jnp.float32),
        mesh=_sc_mesh(),
        scratch_types=[
            pltpu.VMEM((SNCH, SCH), jnp.int32),
            pltpu.VMEM((SNB, SCH, HIDDEN), jnp.float32),
            pltpu.VMEM_SHARED((A_PAD, HIDDEN), jnp.float32),
            [pltpu.SemaphoreType.DMA] * SNB,
        ],
    )


def _segment_sum(h, idx3, zeros):
    return _build_segsum()(h, idx3, zeros)


def _lane_swap_body(shift, x_ref, o_ref):
    x = x_ref[...]
    fwd = pltpu.roll(x, 128 - shift, axis=1)
    bwd = pltpu.roll(x, shift, axis=1)
    lane = lax.broadcasted_iota(jnp.int32, x.shape, 1)
    even = (lane // shift) % 2 == 0
    o_ref[...] = jnp.where(even, fwd, bwd)


def _lane_swap(x, shift):
    rows = x.shape[0]
    blk = 1000
    return pl.pallas_call(
        functools.partial(_lane_swap_body, shift),
        grid=(rows // blk,),
        in_specs=[pl.BlockSpec((blk, 128), lambda i: (i, 0))],
        out_specs=pl.BlockSpec((blk, 128), lambda i: (i, 0)),
        out_shape=jax.ShapeDtypeStruct(x.shape, x.dtype),
    )(x)


def _mm_small_body(x_ref, w_ref, o_ref):
    o_ref[...] = jnp.dot(x_ref[...], w_ref[...],
                         preferred_element_type=jnp.float32)


def _mm_small(x, w):
    grid = A_PAD // ATOM_BLK
    return pl.pallas_call(
        _mm_small_body,
        grid=(grid,),
        in_specs=[pl.BlockSpec((ATOM_BLK, HIDDEN), lambda i: (i, 0)),
                  pl.BlockSpec((HIDDEN, HIDDEN), lambda i: (0, 0))],
        out_specs=pl.BlockSpec((ATOM_BLK, HIDDEN), lambda i: (i, 0)),
        out_shape=jax.ShapeDtypeStruct((A_PAD, HIDDEN), jnp.float32),
    )(x, w)


def _mm_partial_body(p_ref, w_ref, o_ref):
    x = p_ref[0] + p_ref[1]
    o_ref[...] = jnp.dot(x, w_ref[...], preferred_element_type=jnp.float32)


def _mm_partials(p, w):
    grid = A_PAD // ATOM_BLK
    return pl.pallas_call(
        _mm_partial_body,
        grid=(grid,),
        in_specs=[pl.BlockSpec((NC, ATOM_BLK, HIDDEN), lambda i: (0, i, 0)),
                  pl.BlockSpec((HIDDEN, HIDDEN), lambda i: (0, 0))],
        out_specs=pl.BlockSpec((ATOM_BLK, HIDDEN), lambda i: (i, 0)),
        out_shape=jax.ShapeDtypeStruct((A_PAD, HIDDEN), jnp.float32),
    )(p, w)


def _h0_body(g_ref, bf_ref, w_ref, o_ref):
    x = bf_ref[...]
    up = pltpu.roll(x, EDGE_BLK - 1, axis=1)
    dn = pltpu.roll(x, 1, axis=1)
    lane = lax.broadcasted_iota(jnp.int32, x.shape, 1)
    xs = jnp.where(lane % 2 == 0, up, dn)
    mm = lax.dot_general(xs, w_ref[...], (((0,), (0,)), ((), ())),
                         preferred_element_type=jnp.float32)
    o_ref[...] = jnp.maximum(g_ref[...] + mm, 0.0)


def _h0_kernel(g0, bfT, w_ib):
    grid = N_BONDS // EDGE_BLK
    bd = bfT.shape[0]
    return pl.pallas_call(
        _h0_body,
        grid=(grid,),
        in_specs=[pl.BlockSpec((EDGE_BLK, HIDDEN), lambda i: (i, 0)),
                  pl.BlockSpec((bd, EDGE_BLK), lambda i: (0, i)),
                  pl.BlockSpec((bd, HIDDEN), lambda i: (0, 0))],
        out_specs=pl.BlockSpec((EDGE_BLK, HIDDEN), lambda i: (i, 0)),
        out_shape=jax.ShapeDtypeStruct((N_BONDS, HIDDEN), jnp.float32),
    )(g0, bfT, w_ib)


def _update_body(h0_ref, g_ref, h_ref, w_ref, o_ref):
    hw = jnp.dot(h_ref[...], w_ref[...], preferred_element_type=jnp.float32)
    o_ref[...] = jnp.maximum(h0_ref[...] + g_ref[...] - hw, 0.0)


def _update_kernel(h0r, g, hr, w_h):
    grid = N_BONDS // EDGE_BLK
    spec = pl.BlockSpec((EDGE_BLK, HIDDEN), lambda i: (i, 0))
    return pl.pallas_call(
        _update_body,
        grid=(grid,),
        in_specs=[spec, spec, spec,
                  pl.BlockSpec((HIDDEN, HIDDEN), lambda i: (0, 0))],
        out_specs=pl.BlockSpec((EDGE_BLK, HIDDEN), lambda i: (i, 0)),
        out_shape=jax.ShapeDtypeStruct((N_BONDS, HIDDEN), jnp.float32),
    )(h0r, g, hr, w_h)


def _final_body(af_ref, p_ref, a2m_ref, woa_ref, wob_ref, bo_ref,
                mf_ref, w1a_ref, w1b_ref, b1_ref, w2_ref, b2_ref,
                o_ref, msum_ref, cnt_ref):
    i = pl.program_id(0)

    @pl.when(i == 0)
    def _():
        msum_ref[...] = jnp.zeros_like(msum_ref)
        cnt_ref[...] = jnp.zeros_like(cnt_ref)

    m = p_ref[0] + p_ref[1]
    ah = jnp.dot(af_ref[...], woa_ref[...], preferred_element_type=jnp.float32)
    ah = ah + jnp.dot(m, wob_ref[...], preferred_element_type=jnp.float32)
    ah = jnp.maximum(ah + bo_ref[...], 0.0)
    ids = a2m_ref[0]
    onehot = (lax.broadcasted_iota(jnp.int32, (MOL_PAD, ATOM_BLK), 0)
              == ids).astype(jnp.float32)
    msum_ref[...] += jnp.dot(onehot, ah, preferred_element_type=jnp.float32)
    cnt_ref[...] += jnp.dot(onehot,
                            jnp.ones((ATOM_BLK, HIDDEN), jnp.float32),
                            preferred_element_type=jnp.float32)

    @pl.when(i == pl.num_programs(0) - 1)
    def _():
        mol = msum_ref[...] / jnp.maximum(cnt_ref[...], 1.0)
        x = jnp.dot(mol, w1a_ref[...], preferred_element_type=jnp.float32)
        x = x + jnp.dot(mf_ref[...], w1b_ref[...],
                        preferred_element_type=jnp.float32)
        x = jax.nn.sigmoid(x + b1_ref[...])
        o = jnp.dot(x, w2_ref[...], preferred_element_type=jnp.float32)
        o_ref[...] = jax.nn.sigmoid(o + b2_ref[...])


def _final_kernel(af, partials, a2m3, w_oa, w_ob, b_o2, mfp, w1a, w1bp,
                  b12, w2p, b22, ffn_hidden):
    grid = A_PAD // ATOM_BLK
    const = lambda shape: pl.BlockSpec(shape, lambda i: tuple(0 for _ in shape))
    return pl.pallas_call(
        _final_body,
        grid=(grid,),
        in_specs=[
            pl.BlockSpec((ATOM_BLK, HIDDEN), lambda i: (i, 0)),
            pl.BlockSpec((NC, ATOM_BLK, HIDDEN), lambda i: (0, i, 0)),
            pl.BlockSpec((1, 1, ATOM_BLK), lambda i: (i, 0, 0)),
            const((HIDDEN, HIDDEN)),
            const((HIDDEN, HIDDEN)),
            const((1, HIDDEN)),
            const((MOL_PAD, MOL_PAD)),
            const((HIDDEN, ffn_hidden)),
            const((MOL_PAD, ffn_hidden)),
            const((1, ffn_hidden)),
            const((ffn_hidden, HIDDEN)),
            const((1, HIDDEN)),
        ],
        out_specs=pl.BlockSpec((MOL_PAD, HIDDEN), lambda i: (0, 0)),
        out_shape=jax.ShapeDtypeStruct((MOL_PAD, HIDDEN), jnp.float32),
        scratch_shapes=[pltpu.VMEM((MOL_PAD, HIDDEN), jnp.float32),
                        pltpu.VMEM((MOL_PAD, HIDDEN), jnp.float32)],
    )(af, partials, a2m3, w_oa, w_ob, b_o2, mfp, w1a, w1bp, b12, w2p, b22)


def kernel(atom_features, bond_features, bond_index, molecule_features,
           atom_to_molecule, W_i, W_h, W_o, b_o, W1, b1, W2, b2):
    f32 = jnp.float32
    n_mol, feat_dim = molecule_features.shape
    ffn_hidden = W1.shape[1]

    bi = bond_index.astype(jnp.int32).reshape(2 * N_BONDS // 128, 128)
    bir = _lane_swap(bi, 1)
    srcr = bir[:N_BONDS // 128].reshape(NW, GNCH, GCH)
    dstr = bir[N_BONDS // 128:].reshape(NW, SNCH, SCH)

    a2m_pad = jnp.full((A_PAD,), MOL_PAD - 1, jnp.int32).at[:N_ATOMS].set(
        atom_to_molecule.astype(jnp.int32))
    a2m3 = a2m_pad.reshape(A_PAD // ATOM_BLK, 1, ATOM_BLK)
    zeros_atoms = jnp.zeros((A_PAD, HIDDEN), f32)
    af_pad = jnp.zeros((A_PAD, atom_features.shape[1]), f32).at[:N_ATOMS].set(
        atom_features)

    atom_dim = atom_features.shape[1]
    W_ia = W_i[:atom_dim]
    W_ib = W_i[atom_dim:]
    W_oa = W_o[:atom_dim]
    W_ob = W_o[atom_dim:]
    W1a = W1[:HIDDEN]
    W1b = jnp.zeros((MOL_PAD, ffn_hidden), f32).at[:feat_dim].set(W1[HIDDEN:])
    mfp = jnp.zeros((MOL_PAD, MOL_PAD), f32).at[:n_mol, :feat_dim].set(
        molecule_features)
    W2p = jnp.zeros((ffn_hidden, HIDDEN), f32).at[:, :1].set(W2)
    b22 = jnp.zeros((1, HIDDEN), f32).at[0, 0].set(b2[0])
    b_o2 = b_o.reshape(1, HIDDEN)
    b12 = b1.reshape(1, ffn_hidden)

    afW = _mm_small(af_pad, W_ia)
    g0 = _gather_rows(afW, srcr)
    hr = _h0_kernel(g0, bond_features.T, W_ib)
    h0r = hr

    for _ in range(DEPTH - 1):
        partials = _segment_sum(hr, dstr, zeros_atoms)
        amW = _mm_partials(partials, W_h)
        g = _gather_rows(amW, srcr)
        hr = _update_kernel(h0r, g, hr, W_h)

    partials = _segment_sum(hr, dstr, zeros_atoms)
    out = _final_kernel(af_pad, partials, a2m3, W_oa, W_ob, b_o2,
                        mfp, W1a, W1b, b12, W2p, b22, ffn_hidden)
    return out[:n_mol, :1]


# speedup vs baseline: 3.3858x; 1.0399x over previous
import functools

import jax
import jax.numpy as jnp
from jax import lax
from jax.experimental import pallas as pl
from jax.experimental.pallas import tpu as pltpu
from jax.experimental.pallas import tpu_sc as plsc

N_ATOMS = 10000
N_BONDS = 320000
HIDDEN = 128
DEPTH = 3

NC = 2
NS = 16
NW = NC * NS
EH = N_BONDS // 2
EPW = EH // NW
CH = 40
NCH = EPW // CH
GNB = 5
SNB = 2
A_PAD = 10240

ATOM_BLK = 1024
EDGE_BLK = 6400
HBLK = EH // EDGE_BLK
MOL_PAD = 256

assert EH % EDGE_BLK == 0 and A_PAD % ATOM_BLK == 0
assert EPW == NCH * CH and CH % 8 == 0 and CH <= 128
assert NCH % GNB == 0


def _sc_mesh():
    return plsc.VectorSubcoreMesh(
        core_axis_name="c", subcore_axis_name="s",
        num_cores=NC, num_subcores=NS)


def _gather_body(table_hbm, idx_hbm, out_hbm, idx_v, bufs, sems):
    c = lax.axis_index("c")
    s = lax.axis_index("s")
    w = c * NS + s
    pltpu.sync_copy(idx_hbm.at[w], idx_v)

    def body(i, carry):
        j0 = i * GNB
        for k in range(GNB):
            pltpu.async_copy(table_hbm.at[idx_v.at[j0 + k]], bufs.at[k],
                             sems[k])
        for k in range(GNB):
            pltpu.make_async_copy(table_hbm.at[idx_v.at[j0 + k]], bufs.at[k],
                                  sems[k]).wait()
            pltpu.sync_copy(
                bufs.at[k],
                out_hbm.at[pl.ds(w * EPW + (j0 + k) * CH, CH)])
        return carry

    lax.fori_loop(0, NCH // GNB, body, 0)


@functools.cache
def _build_gather():
    return pl.kernel(
        _gather_body,
        out_type=jax.ShapeDtypeStruct((EH, HIDDEN), jnp.float32),
        mesh=_sc_mesh(),
        scratch_types=[
            pltpu.VMEM((NCH, CH), jnp.int32),
            pltpu.VMEM((GNB, CH, HIDDEN), jnp.float32),
            [pltpu.SemaphoreType.DMA] * GNB,
        ],
    )


def _gather_rows(table, idx3):
    return _build_gather()(table, idx3)


def _segsum_body(h_hbm, idx_hbm, zeros_hbm, out_hbm, idx_v, rows_v, acc_sh,
                 sems):
    c = lax.axis_index("c")
    s = lax.axis_index("s")
    w = c * NS + s
    rows_per_tile = A_PAD // NS
    pltpu.sync_copy(zeros_hbm.at[pl.ds(s * rows_per_tile, rows_per_tile)],
                    acc_sh.at[pl.ds(s * rows_per_tile, rows_per_tile)])
    pltpu.sync_copy(idx_hbm.at[w], idx_v)
    plsc.subcore_barrier()

    def step(j, k):
        pltpu.make_async_copy(
            h_hbm.at[pl.ds(w * EPW + j * CH, CH)], rows_v.at[k],
            sems[k]).wait()
        pltpu.sync_copy(rows_v.at[k], acc_sh.at[idx_v.at[j]], add=True)

    def start(j, k):
        pltpu.async_copy(h_hbm.at[pl.ds(w * EPW + j * CH, CH)],
                         rows_v.at[k], sems[k])

    def body(i, carry):
        j0 = i * SNB
        for k in range(SNB):
            start(j0 + k, k)
        for k in range(SNB):
            step(j0 + k, k)
        return carry

    pairs = (NCH - 1) // SNB
    lax.fori_loop(0, pairs, body, 0)
    for j in range(pairs * SNB, NCH):
        start(j, 0)
        step(j, 0)
    plsc.subcore_barrier()
    pltpu.sync_copy(acc_sh.at[pl.ds(s * rows_per_tile, rows_per_tile)],
                    out_hbm.at[c, pl.ds(s * rows_per_tile, rows_per_tile)])


@functools.cache
def _build_segsum():
    return pl.kernel(
        _segsum_body,
        out_type=jax.ShapeDtypeStruct((NC, A_PAD, HIDDEN), jnp.float32),
        mesh=_sc_mesh(),
        scratch_types=[
            pltpu.VMEM((NCH, CH), jnp.int32),
            pltpu.VMEM((SNB, CH, HIDDEN), jnp.float32),
            pltpu.VMEM_SHARED((A_PAD, HIDDEN), jnp.float32),
            [pltpu.SemaphoreType.DMA] * SNB,
        ],
    )


def _segment_sum(h, idx3, zeros):
    return _build_segsum()(h, idx3, zeros)


def _lane_swap_body(shift, x_ref, o_ref):
    x = x_ref[...]
    fwd = pltpu.roll(x, 128 - shift, axis=1)
    bwd = pltpu.roll(x, shift, axis=1)
    lane = lax.broadcasted_iota(jnp.int32, x.shape, 1)
    even = (lane // shift) % 2 == 0
    o_ref[...] = jnp.where(even, fwd, bwd)


def _lane_swap(x, shift):
    rows = x.shape[0]
    blk = 1000
    return pl.pallas_call(
        functools.partial(_lane_swap_body, shift),
        grid=(rows // blk,),
        in_specs=[pl.BlockSpec((blk, 128), lambda i: (i, 0))],
        out_specs=pl.BlockSpec((blk, 128), lambda i: (i, 0)),
        out_shape=jax.ShapeDtypeStruct(x.shape, x.dtype),
    )(x)


def _mm_small_body(x_ref, w_ref, o_ref):
    o_ref[...] = jnp.dot(x_ref[...], w_ref[...],
                         preferred_element_type=jnp.float32)


def _mm_small(x, w):
    grid = A_PAD // ATOM_BLK
    return pl.pallas_call(
        _mm_small_body,
        grid=(grid,),
        in_specs=[pl.BlockSpec((ATOM_BLK, HIDDEN), lambda i: (i, 0)),
                  pl.BlockSpec((HIDDEN, HIDDEN), lambda i: (0, 0))],
        out_specs=pl.BlockSpec((ATOM_BLK, HIDDEN), lambda i: (i, 0)),
        out_shape=jax.ShapeDtypeStruct((A_PAD, HIDDEN), jnp.float32),
    )(x, w)


def _mm_partial_body(pa_ref, pb_ref, w_ref, o_ref):
    x = pa_ref[0] + pa_ref[1] + pb_ref[0] + pb_ref[1]
    o_ref[...] = jnp.dot(x, w_ref[...], preferred_element_type=jnp.float32)


def _mm_partials(pa, pb, w):
    grid = A_PAD // ATOM_BLK
    pspec = pl.BlockSpec((NC, ATOM_BLK, HIDDEN), lambda i: (0, i, 0))
    return pl.pallas_call(
        _mm_partial_body,
        grid=(grid,),
        in_specs=[pspec, pspec,
                  pl.BlockSpec((HIDDEN, HIDDEN), lambda i: (0, 0))],
        out_specs=pl.BlockSpec((ATOM_BLK, HIDDEN), lambda i: (i, 0)),
        out_shape=jax.ShapeDtypeStruct((A_PAD, HIDDEN), jnp.float32),
    )(pa, pb, w)


def _h0_body(g_ref, bf_ref, w_ref, o_ref):
    x = bf_ref[...]
    up = pltpu.roll(x, EDGE_BLK - 1, axis=1)
    dn = pltpu.roll(x, 1, axis=1)
    lane = lax.broadcasted_iota(jnp.int32, x.shape, 1)
    xs = jnp.where(lane % 2 == 0, up, dn)
    mm = lax.dot_general(xs, w_ref[...], (((0,), (0,)), ((), ())),
                         preferred_element_type=jnp.float32)
    o_ref[...] = jnp.maximum(g_ref[...] + mm, 0.0)


def _h0_kernel(g0, bfT, w_ib, half):
    bd = bfT.shape[0]
    off = half * HBLK
    return pl.pallas_call(
        _h0_body,
        grid=(HBLK,),
        in_specs=[pl.BlockSpec((EDGE_BLK, HIDDEN), lambda i: (i, 0)),
                  pl.BlockSpec((bd, EDGE_BLK), lambda i: (0, i + off)),
                  pl.BlockSpec((bd, HIDDEN), lambda i: (0, 0))],
        out_specs=pl.BlockSpec((EDGE_BLK, HIDDEN), lambda i: (i, 0)),
        out_shape=jax.ShapeDtypeStruct((EH, HIDDEN), jnp.float32),
    )(g0, bfT, w_ib)


def _update_body(h0_ref, g_ref, h_ref, w_ref, o_ref):
    hw = jnp.dot(h_ref[...], w_ref[...], preferred_element_type=jnp.float32)
    o_ref[...] = jnp.maximum(h0_ref[...] + g_ref[...] - hw, 0.0)


def _update_kernel(h0r, g, hr, w_h):
    spec = pl.BlockSpec((EDGE_BLK, HIDDEN), lambda i: (i, 0))
    return pl.pallas_call(
        _update_body,
        grid=(HBLK,),
        in_specs=[spec, spec, spec,
                  pl.BlockSpec((HIDDEN, HIDDEN), lambda i: (0, 0))],
        out_specs=pl.BlockSpec((EDGE_BLK, HIDDEN), lambda i: (i, 0)),
        out_shape=jax.ShapeDtypeStruct((EH, HIDDEN), jnp.float32),
    )(h0r, g, hr, w_h)


def _final_body(af_ref, pa_ref, pb_ref, a2m_ref, woa_ref, wob_ref, bo_ref,
                mf_ref, w1a_ref, w1b_ref, b1_ref, w2_ref, b2_ref,
                o_ref, msum_ref, cnt_ref):
    i = pl.program_id(0)

    @pl.when(i == 0)
    def _():
        msum_ref[...] = jnp.zeros_like(msum_ref)
        cnt_ref[...] = jnp.zeros_like(cnt_ref)

    m = pa_ref[0] + pa_ref[1] + pb_ref[0] + pb_ref[1]
    ah = jnp.dot(af_ref[...], woa_ref[...], preferred_element_type=jnp.float32)
    ah = ah + jnp.dot(m, wob_ref[...], preferred_element_type=jnp.float32)
    ah = jnp.maximum(ah + bo_ref[...], 0.0)
    ids = a2m_ref[0]
    onehot = (lax.broadcasted_iota(jnp.int32, (MOL_PAD, ATOM_BLK), 0)
              == ids).astype(jnp.float32)
    msum_ref[...] += jnp.dot(onehot, ah, preferred_element_type=jnp.float32)
    cnt_ref[...] += jnp.dot(onehot,
                            jnp.ones((ATOM_BLK, HIDDEN), jnp.float32),
                            preferred_element_type=jnp.float32)

    @pl.when(i == pl.num_programs(0) - 1)
    def _():
        mol = msum_ref[...] / jnp.maximum(cnt_ref[...], 1.0)
        x = jnp.dot(mol, w1a_ref[...], preferred_element_type=jnp.float32)
        x = x + jnp.dot(mf_ref[...], w1b_ref[...],
                        preferred_element_type=jnp.float32)
        x = jax.nn.sigmoid(x + b1_ref[...])
        o = jnp.dot(x, w2_ref[...], preferred_element_type=jnp.float32)
        o_ref[...] = jax.nn.sigmoid(o + b2_ref[...])


def _final_kernel(af, pa, pb, a2m3, w_oa, w_ob, b_o2, mfp, w1a, w1bp,
                  b12, w2p, b22, ffn_hidden):
    grid = A_PAD // ATOM_BLK
    const = lambda shape: pl.BlockSpec(shape, lambda i: tuple(0 for _ in shape))
    pspec = pl.BlockSpec((NC, ATOM_BLK, HIDDEN), lambda i: (0, i, 0))
    return pl.pallas_call(
        _final_body,
        grid=(grid,),
        in_specs=[
            pl.BlockSpec((ATOM_BLK, HIDDEN), lambda i: (i, 0)),
            pspec,
            pspec,
            pl.BlockSpec((1, 1, ATOM_BLK), lambda i: (i, 0, 0)),
            const((HIDDEN, HIDDEN)),
            const((HIDDEN, HIDDEN)),
            const((1, HIDDEN)),
            const((MOL_PAD, MOL_PAD)),
            const((HIDDEN, ffn_hidden)),
            const((MOL_PAD, ffn_hidden)),
            const((1, ffn_hidden)),
            const((ffn_hidden, HIDDEN)),
            const((1, HIDDEN)),
        ],
        out_specs=pl.BlockSpec((MOL_PAD, HIDDEN), lambda i: (0, 0)),
        out_shape=jax.ShapeDtypeStruct((MOL_PAD, HIDDEN), jnp.float32),
        scratch_shapes=[pltpu.VMEM((MOL_PAD, HIDDEN), jnp.float32),
                        pltpu.VMEM((MOL_PAD, HIDDEN), jnp.float32)],
    )(af, pa, pb, a2m3, w_oa, w_ob, b_o2, mfp, w1a, w1bp, b12, w2p, b22)


def kernel(atom_features, bond_features, bond_index, molecule_features,
           atom_to_molecule, W_i, W_h, W_o, b_o, W1, b1, W2, b2):
    f32 = jnp.float32
    n_mol, feat_dim = molecule_features.shape
    ffn_hidden = W1.shape[1]

    bi = bond_index.astype(jnp.int32).reshape(2 * N_BONDS // 128, 128)
    bir = _lane_swap(bi, 1)
    q = N_BONDS // 128 // 2
    srcrA = bir[:q].reshape(NW, NCH, CH)
    srcrB = bir[q:2 * q].reshape(NW, NCH, CH)
    dstrA = bir[2 * q:3 * q].reshape(NW, NCH, CH)
    dstrB = bir[3 * q:].reshape(NW, NCH, CH)
    a2m_pad = jnp.full((A_PAD,), MOL_PAD - 1, jnp.int32).at[:N_ATOMS].set(
        atom_to_molecule.astype(jnp.int32))
    a2m3 = a2m_pad.reshape(A_PAD // ATOM_BLK, 1, ATOM_BLK)
    zeros_atoms = jnp.zeros((A_PAD, HIDDEN), f32)
    af_pad = jnp.zeros((A_PAD, atom_features.shape[1]), f32).at[:N_ATOMS].set(
        atom_features)

    atom_dim = atom_features.shape[1]
    W_ia = W_i[:atom_dim]
    W_ib = W_i[atom_dim:]
    W_oa = W_o[:atom_dim]
    W_ob = W_o[atom_dim:]
    W1a = W1[:HIDDEN]
    W1b = jnp.zeros((MOL_PAD, ffn_hidden), f32).at[:feat_dim].set(W1[HIDDEN:])
    mfp = jnp.zeros((MOL_PAD, MOL_PAD), f32).at[:n_mol, :feat_dim].set(
        molecule_features)
    W2p = jnp.zeros((ffn_hidden, HIDDEN), f32).at[:, :1].set(W2)
    b22 = jnp.zeros((1, HIDDEN), f32).at[0, 0].set(b2[0])
    b_o2 = b_o.reshape(1, HIDDEN)
    b12 = b1.reshape(1, ffn_hidden)

    afW = _mm_small(af_pad, W_ia)
    bfT = bond_features.T
    hA = _h0_kernel(_gather_rows(afW, srcrA), bfT, W_ib, 0)
    hB = _h0_kernel(_gather_rows(afW, srcrB), bfT, W_ib, 1)
    h0A, h0B = hA, hB

    for _ in range(DEPTH - 1):
        pA = _segment_sum(hA, dstrA, zeros_atoms)
        pB = _segment_sum(hB, dstrB, zeros_atoms)
        amW = _mm_partials(pA, pB, W_h)
        gA = _gather_rows(amW, srcrA)
        gB = _gather_rows(amW, srcrB)
        hA = _update_kernel(h0A, gA, hA, W_h)
        hB = _update_kernel(h0B, gB, hB, W_h)

    pA = _segment_sum(hA, dstrA, zeros_atoms)
    pB = _segment_sum(hB, dstrB, zeros_atoms)
    out = _final_kernel(af_pad, pA, pB, a2m3, W_oa, W_ob, b_o2,
                        mfp, W1a, W1b, b12, W2p, b22, ffn_hidden)
    return out[:n_mol, :1]
